# baseline (speedup 1.0000x reference)
import functools

import numpy as np
import jax
import jax.numpy as jnp
from jax.experimental import pallas as pl
from jax.experimental.pallas import tpu as pltpu

_MAX_SEQ_LEN = 4096
_BATCH = 2
_FROM_SEQ = 2048
_TO_SEQ = 2048
_D_MODEL = 1024
_NUM_HEADS = 16
_HEAD = 64
_BLK = 64
_NUM_RAND = 3
_NROWS = _FROM_SEQ // _BLK
_NCOLS = _TO_SEQ // _BLK
_NSLOT = 7
_NH = _NUM_HEADS * _HEAD
_SCALE = 0.125


def _block_rand_mask(from_seq_length, to_seq_length, from_block_size,
                     to_block_size, num_rand_blocks, last_idx=-1):
    rand_attn = np.zeros(
        (from_seq_length // from_block_size - 2, num_rand_blocks), dtype=np.int32)
    middle_seq = np.arange(1, to_seq_length // to_block_size - 1, dtype=np.int32)
    last = to_seq_length // to_block_size - 1
    if last_idx > 2 * to_block_size:
        last = last_idx // to_block_size - 1
    r = num_rand_blocks
    for i in range(1, from_seq_length // from_block_size - 1):
        start = i - 2
        end = i
        if i == 1:
            rand_attn[i - 1, :] = np.random.permutation(middle_seq[2:last])[:r]
        elif i == 2:
            rand_attn[i - 1, :] = np.random.permutation(middle_seq[3:last])[:r]
        elif i == from_seq_length // from_block_size - 3:
            rand_attn[i - 1, :] = np.random.permutation(middle_seq[:last])[:r]
        elif i == from_seq_length // from_block_size - 2:
            rand_attn[i - 1, :] = np.random.permutation(middle_seq[:last])[:r]
        elif start > last:
            start = last
            rand_attn[i - 1, :] = np.random.permutation(middle_seq[:start])[:r]
        elif end + 1 == last:
            rand_attn[i - 1, :] = np.random.permutation(middle_seq[:start])[:r]
        else:
            rand_attn[i - 1, :] = np.random.permutation(
                np.concatenate((middle_seq[:start], middle_seq[end + 1:last])))[:r]
    return rand_attn


@functools.lru_cache(maxsize=1)
def _block_table():
    np.random.seed(0)
    rand_attn = _block_rand_mask(_MAX_SEQ_LEN, _MAX_SEQ_LEN, _BLK, _BLK,
                                 _NUM_RAND, last_idx=1024)
    idx = np.zeros((_NROWS, _NSLOT), dtype=np.int32)
    cnt = np.zeros((_NROWS,), dtype=np.int32)
    cnt[0] = _NCOLS
    for i in range(1, _NROWS):
        blocks = {0}
        for j in (i - 1, i, i + 1):
            if 0 <= j < _NCOLS:
                blocks.add(j)
        for j in rand_attn[i - 1]:
            if int(j) < _NCOLS:
                blocks.add(int(j))
        blist = sorted(blocks)
        assert len(blist) <= _NSLOT
        cnt[i] = len(blist)
        for s, j in enumerate(blist):
            idx[i, s] = j
    return idx, cnt


def _store_heads(dst_ref, acc):
    for n in range(_NUM_HEADS):
        lo = n * _HEAD
        dst_ref[n] = acc[:, lo:lo + _HEAD]


def _store_ctx_flat(o_ref, ctx):
    for n in range(_NUM_HEADS):
        lo = n * _HEAD
        o_ref[0, :, lo:lo + _HEAD] = ctx[n]


def _softmax_pv(sc, v):
    m = jnp.max(sc, axis=-1, keepdims=True)
    e = jnp.exp(sc - m)
    denom = jnp.sum(e, axis=-1, keepdims=True)
    ctx = jax.lax.dot_general(
        e.astype(jnp.bfloat16), v,
        dimension_numbers=(((2,), (1,)), ((0,), (0,))),
        preferred_element_type=jnp.float32)
    return ctx / denom


def _fused_kernel(idx_ref, cnt_ref, xf_ref, xt_ref, wq_ref, wk_ref, wv_ref,
                  bq_ref, bk_ref, bv_ref, o_ref,
                  ks_ref, vs_ref, wqs_ref, kg_ref, vg_ref):
    i = pl.program_id(1)

    @pl.when(i == 0)
    def _project():
        xt = xt_ref[0].astype(jnp.bfloat16)
        wk = wk_ref[...].astype(jnp.bfloat16)
        acck = jnp.dot(xt, wk, preferred_element_type=jnp.float32)
        _store_heads(ks_ref, (acck + bk_ref[...]).astype(jnp.bfloat16))
        wv = wv_ref[...].astype(jnp.bfloat16)
        accv = jnp.dot(xt, wv, preferred_element_type=jnp.float32)
        _store_heads(vs_ref, (accv + bv_ref[...]).astype(jnp.bfloat16))
        wqs_ref[...] = (wq_ref[...] * _SCALE).astype(jnp.bfloat16)

    @pl.when(i > 0)
    def _attend():
        xq = xf_ref[0].astype(jnp.bfloat16)
        qflat = jnp.dot(xq, wqs_ref[...], preferred_element_type=jnp.float32)
        qflat = (qflat + bq_ref[...] * _SCALE).astype(jnp.bfloat16)
        q = jnp.stack(
            [qflat[:, n * _HEAD:(n + 1) * _HEAD] for n in range(_NUM_HEADS)])

        @pl.when(i == 1)
        def _dense():
            sc = jax.lax.dot_general(
                q, ks_ref[...],
                dimension_numbers=(((2,), (2,)), ((0,), (0,))),
                preferred_element_type=jnp.float32)
            _store_ctx_flat(o_ref, _softmax_pv(sc, vs_ref[...]))

        @pl.when(i > 1)
        def _sparse():
            r = i - 1
            for s in range(_NSLOT):
                j = idx_ref[r, s]
                kg_ref[:, pl.ds(s * _BLK, _BLK), :] = \
                    ks_ref[:, pl.ds(j * _BLK, _BLK), :]
                vg_ref[:, pl.ds(s * _BLK, _BLK), :] = \
                    vs_ref[:, pl.ds(j * _BLK, _BLK), :]
            sc = jax.lax.dot_general(
                q, kg_ref[...],
                dimension_numbers=(((2,), (2,)), ((0,), (0,))),
                preferred_element_type=jnp.float32)
            cnt = cnt_ref[r]
            col = jax.lax.broadcasted_iota(jnp.int32, sc.shape, 2)
            sc = jnp.where(col < cnt * _BLK, sc, -1e30)
            _store_ctx_flat(o_ref, _softmax_pv(sc, vg_ref[...]))


def kernel(from_tensor, to_tensor, W_q, W_k, W_v, b_q, b_k, b_v):
    idx, cnt = _block_table()
    wq = W_q.reshape(_D_MODEL, _NH)
    wk = W_k.reshape(_D_MODEL, _NH)
    wv = W_v.reshape(_D_MODEL, _NH)
    bq = b_q.reshape(1, _NH)
    bk = b_k.reshape(1, _NH)
    bv = b_v.reshape(1, _NH)

    def row(i):
        return jnp.maximum(i - 1, 0)

    grid_spec = pltpu.PrefetchScalarGridSpec(
        num_scalar_prefetch=2,
        grid=(_BATCH, _NROWS + 1),
        in_specs=[
            pl.BlockSpec((1, _BLK, _D_MODEL),
                         lambda b, i, *_: (b, row(i), 0)),
            pl.BlockSpec((1, _TO_SEQ, _D_MODEL),
                         lambda b, i, *_: (b, 0, 0)),
            pl.BlockSpec((_D_MODEL, _NH), lambda b, i, *_: (0, 0)),
            pl.BlockSpec((_D_MODEL, _NH), lambda b, i, *_: (0, 0)),
            pl.BlockSpec((_D_MODEL, _NH), lambda b, i, *_: (0, 0)),
            pl.BlockSpec((1, _NH), lambda b, i, *_: (0, 0)),
            pl.BlockSpec((1, _NH), lambda b, i, *_: (0, 0)),
            pl.BlockSpec((1, _NH), lambda b, i, *_: (0, 0)),
        ],
        out_specs=pl.BlockSpec((1, _BLK, _NH), lambda b, i, *_: (b, row(i), 0)),
        scratch_shapes=[
            pltpu.VMEM((_NUM_HEADS, _TO_SEQ, _HEAD), jnp.bfloat16),
            pltpu.VMEM((_NUM_HEADS, _TO_SEQ, _HEAD), jnp.bfloat16),
            pltpu.VMEM((_D_MODEL, _NH), jnp.bfloat16),
            pltpu.VMEM((_NUM_HEADS, _NSLOT * _BLK, _HEAD), jnp.bfloat16),
            pltpu.VMEM((_NUM_HEADS, _NSLOT * _BLK, _HEAD), jnp.bfloat16),
        ],
    )
    ctx = pl.pallas_call(
        _fused_kernel,
        grid_spec=grid_spec,
        out_shape=jax.ShapeDtypeStruct((_BATCH, _FROM_SEQ, _NH), jnp.float32),
        compiler_params=pltpu.CompilerParams(
            dimension_semantics=("parallel", "arbitrary")),
    )(jnp.asarray(idx), jnp.asarray(cnt),
      from_tensor, to_tensor, wq, wk, wv, bq, bk, bv)
    return ctx.reshape(_BATCH, _FROM_SEQ, _NUM_HEADS, _HEAD)


# speedup vs baseline: 1.0108x; 1.0108x over previous
import functools

import numpy as np
import jax
import jax.numpy as jnp
from jax.experimental import pallas as pl
from jax.experimental.pallas import tpu as pltpu

_MAX_SEQ_LEN = 4096
_BATCH = 2
_FROM_SEQ = 2048
_TO_SEQ = 2048
_D_MODEL = 1024
_NUM_HEADS = 16
_HEAD = 64
_BLK = 64
_NUM_RAND = 3
_NROWS = _FROM_SEQ // _BLK
_NCOLS = _TO_SEQ // _BLK
_NSLOT = 7
_NH = _NUM_HEADS * _HEAD
_SCALE = 0.125


def _block_rand_mask(from_seq_length, to_seq_length, from_block_size,
                     to_block_size, num_rand_blocks, last_idx=-1):
    rand_attn = np.zeros(
        (from_seq_length // from_block_size - 2, num_rand_blocks), dtype=np.int32)
    middle_seq = np.arange(1, to_seq_length // to_block_size - 1, dtype=np.int32)
    last = to_seq_length // to_block_size - 1
    if last_idx > 2 * to_block_size:
        last = last_idx // to_block_size - 1
    r = num_rand_blocks
    for i in range(1, from_seq_length // from_block_size - 1):
        start = i - 2
        end = i
        if i == 1:
            rand_attn[i - 1, :] = np.random.permutation(middle_seq[2:last])[:r]
        elif i == 2:
            rand_attn[i - 1, :] = np.random.permutation(middle_seq[3:last])[:r]
        elif i == from_seq_length // from_block_size - 3:
            rand_attn[i - 1, :] = np.random.permutation(middle_seq[:last])[:r]
        elif i == from_seq_length // from_block_size - 2:
            rand_attn[i - 1, :] = np.random.permutation(middle_seq[:last])[:r]
        elif start > last:
            start = last
            rand_attn[i - 1, :] = np.random.permutation(middle_seq[:start])[:r]
        elif end + 1 == last:
            rand_attn[i - 1, :] = np.random.permutation(middle_seq[:start])[:r]
        else:
            rand_attn[i - 1, :] = np.random.permutation(
                np.concatenate((middle_seq[:start], middle_seq[end + 1:last])))[:r]
    return rand_attn


@functools.lru_cache(maxsize=1)
def _block_table():
    np.random.seed(0)
    rand_attn = _block_rand_mask(_MAX_SEQ_LEN, _MAX_SEQ_LEN, _BLK, _BLK,
                                 _NUM_RAND, last_idx=1024)
    idx = np.zeros((_NROWS, _NSLOT), dtype=np.int32)
    cnt = np.zeros((_NROWS,), dtype=np.int32)
    cnt[0] = _NCOLS
    for i in range(1, _NROWS):
        blocks = {0}
        for j in (i - 1, i, i + 1):
            if 0 <= j < _NCOLS:
                blocks.add(j)
        for j in rand_attn[i - 1]:
            if int(j) < _NCOLS:
                blocks.add(int(j))
        blist = sorted(blocks)
        assert len(blist) <= _NSLOT
        cnt[i] = len(blist)
        for s, j in enumerate(blist):
            idx[i, s] = j
    return idx, cnt


def _store_heads(dst_ref, acc):
    for n in range(_NUM_HEADS):
        lo = n * _HEAD
        dst_ref[n] = acc[:, lo:lo + _HEAD]


def _store_ctx_flat(o_ref, ctx):
    for n in range(_NUM_HEADS):
        lo = n * _HEAD
        o_ref[0, :, lo:lo + _HEAD] = ctx[n]


def _softmax_pv(sc, v):
    m = jnp.max(sc, axis=-1, keepdims=True)
    e = jnp.exp(sc - m)
    denom = jnp.sum(e, axis=-1, keepdims=True)
    ctx = jax.lax.dot_general(
        e.astype(jnp.bfloat16), v,
        dimension_numbers=(((2,), (1,)), ((0,), (0,))),
        preferred_element_type=jnp.float32)
    return ctx / denom


def _fused_kernel(idx_ref, cnt_ref, xf_ref, xt_ref, wq_ref, wk_ref, wv_ref,
                  bq_ref, bk_ref, bv_ref, o_ref,
                  ks_ref, vs_ref, kg_ref, vg_ref):
    i = pl.program_id(1)

    @pl.when(i == 0)
    def _project():
        xt = xt_ref[0].astype(jnp.bfloat16)
        acck = jnp.dot(xt, wk_ref[...], preferred_element_type=jnp.float32)
        _store_heads(ks_ref, (acck + bk_ref[...]).astype(jnp.bfloat16))
        accv = jnp.dot(xt, wv_ref[...], preferred_element_type=jnp.float32)
        _store_heads(vs_ref, (accv + bv_ref[...]).astype(jnp.bfloat16))

    @pl.when(i > 0)
    def _attend():
        xq = xf_ref[0].astype(jnp.bfloat16)
        qflat = jnp.dot(xq, wq_ref[...], preferred_element_type=jnp.float32)
        qflat = (qflat + bq_ref[...]).astype(jnp.bfloat16)
        q = jnp.stack(
            [qflat[:, n * _HEAD:(n + 1) * _HEAD] for n in range(_NUM_HEADS)])

        @pl.when(i == 1)
        def _dense():
            sc = jax.lax.dot_general(
                q, ks_ref[...],
                dimension_numbers=(((2,), (2,)), ((0,), (0,))),
                preferred_element_type=jnp.float32)
            _store_ctx_flat(o_ref, _softmax_pv(sc, vs_ref[...]))

        @pl.when(i > 1)
        def _sparse():
            r = i - 1
            for s in range(_NSLOT):
                j = idx_ref[r, s]
                kg_ref[:, pl.ds(s * _BLK, _BLK), :] = \
                    ks_ref[:, pl.ds(j * _BLK, _BLK), :]
                vg_ref[:, pl.ds(s * _BLK, _BLK), :] = \
                    vs_ref[:, pl.ds(j * _BLK, _BLK), :]
            sc = jax.lax.dot_general(
                q, kg_ref[...],
                dimension_numbers=(((2,), (2,)), ((0,), (0,))),
                preferred_element_type=jnp.float32)
            cnt = cnt_ref[r]
            col = jax.lax.broadcasted_iota(jnp.int32, sc.shape, 2)
            sc = jnp.where(col < cnt * _BLK, sc, -1e30)
            _store_ctx_flat(o_ref, _softmax_pv(sc, vg_ref[...]))


def kernel(from_tensor, to_tensor, W_q, W_k, W_v, b_q, b_k, b_v):
    idx, cnt = _block_table()
    bf16 = jnp.bfloat16
    wq = (W_q.reshape(_D_MODEL, _NH) * _SCALE).astype(bf16)
    wk = W_k.reshape(_D_MODEL, _NH).astype(bf16)
    wv = W_v.reshape(_D_MODEL, _NH).astype(bf16)
    bq = b_q.reshape(1, _NH) * _SCALE
    bk = b_k.reshape(1, _NH)
    bv = b_v.reshape(1, _NH)

    def row(i):
        return jnp.maximum(i - 1, 0)

    grid_spec = pltpu.PrefetchScalarGridSpec(
        num_scalar_prefetch=2,
        grid=(_BATCH, _NROWS + 1),
        in_specs=[
            pl.BlockSpec((1, _BLK, _D_MODEL),
                         lambda b, i, *_: (b, row(i), 0)),
            pl.BlockSpec((1, _TO_SEQ, _D_MODEL),
                         lambda b, i, *_: (b, 0, 0)),
            pl.BlockSpec((_D_MODEL, _NH), lambda b, i, *_: (0, 0)),
            pl.BlockSpec((_D_MODEL, _NH), lambda b, i, *_: (0, 0)),
            pl.BlockSpec((_D_MODEL, _NH), lambda b, i, *_: (0, 0)),
            pl.BlockSpec((1, _NH), lambda b, i, *_: (0, 0)),
            pl.BlockSpec((1, _NH), lambda b, i, *_: (0, 0)),
            pl.BlockSpec((1, _NH), lambda b, i, *_: (0, 0)),
        ],
        out_specs=pl.BlockSpec((1, _BLK, _NH), lambda b, i, *_: (b, row(i), 0)),
        scratch_shapes=[
            pltpu.VMEM((_NUM_HEADS, _TO_SEQ, _HEAD), jnp.bfloat16),
            pltpu.VMEM((_NUM_HEADS, _TO_SEQ, _HEAD), jnp.bfloat16),
            pltpu.VMEM((_NUM_HEADS, _NSLOT * _BLK, _HEAD), jnp.bfloat16),
            pltpu.VMEM((_NUM_HEADS, _NSLOT * _BLK, _HEAD), jnp.bfloat16),
        ],
    )
    ctx = pl.pallas_call(
        _fused_kernel,
        grid_spec=grid_spec,
        out_shape=jax.ShapeDtypeStruct((_BATCH, _FROM_SEQ, _NH), jnp.float32),
        compiler_params=pltpu.CompilerParams(
            dimension_semantics=("parallel", "arbitrary")),
    )(jnp.asarray(idx), jnp.asarray(cnt),
      from_tensor, to_tensor, wq, wk, wv, bq, bk, bv)
    return ctx.reshape(_BATCH, _FROM_SEQ, _NUM_HEADS, _HEAD)
